# baseline (speedup 1.0000x reference)
import jax
import jax.numpy as jnp
from jax import lax
from jax.experimental import pallas as pl
from jax.experimental.pallas import tpu as pltpu
from jax.experimental.pallas import tpu_sc as plsc

N = 10000
D = 128
E1 = 320000
E2 = 640000
NS = 16
CHUNK = 64
NBUF = 5
RB = 80
NRC = N // RB
NCH1 = E1 // CHUNK
NCH2 = E2 // CHUNK
NCH2A = 7584
NCH2B = NCH2 - NCH2A


def _spmm_edges(row_hbm, col_hbm, x_hbm, acc, ridx, cidx, gbuf,
                irs, ics, gs, ss, s, ch0, nch):
    kmax = (nch // NS + NBUF) // NBUF
    kmax += kmax % 2
    lim = ch0 + nch

    def chunk(k, b):
        return ch0 + s + (k * NBUF + b) * NS

    def scatter_wait(b):
        pltpu.make_async_copy(gbuf.at[b], acc.at[ridx.at[b]], ss.at[b]).wait()

    def cidx_prefetch(k, q):
        for b in range(NBUF):
            ch = chunk(k, b)

            @pl.when(ch < lim)
            def _():
                i = q * NBUF + b
                pltpu.async_copy(col_hbm.at[pl.ds(ch * CHUNK, CHUNK)],
                                 cidx.at[i], ics.at[i])

    def round_(k, p, drain):
        if drain:
            for b in range(NBUF):
                chp = chunk(k - 1, b)

                @pl.when(chp < lim)
                def _():
                    scatter_wait(b)
        cidx_prefetch(k + 1, 1 - p)
        for b in range(NBUF):
            ch = chunk(k, b)

            @pl.when(ch < lim)
            def _():
                i = p * NBUF + b
                off = ch * CHUNK
                pltpu.async_copy(row_hbm.at[pl.ds(off, CHUNK)], ridx.at[b],
                                 irs.at[b])
                pltpu.make_async_copy(col_hbm.at[pl.ds(off, CHUNK)],
                                      cidx.at[i], ics.at[i]).wait()
                pltpu.async_copy(x_hbm.at[cidx.at[i]], gbuf.at[b], gs.at[b])
        for b in range(NBUF):
            ch = chunk(k, b)

            @pl.when(ch < lim)
            def _():
                i = p * NBUF + b
                off = ch * CHUNK
                pltpu.make_async_copy(row_hbm.at[pl.ds(off, CHUNK)],
                                      ridx.at[b], irs.at[b]).wait()
                pltpu.make_async_copy(x_hbm.at[cidx.at[i]], gbuf.at[b],
                                      gs.at[b]).wait()
                pltpu.async_copy(gbuf.at[b], acc.at[ridx.at[b]], ss.at[b],
                                 add=True)

    def body(m, carry):
        k = m * 2
        round_(k, 0, drain=True)
        round_(k + 1, 1, drain=True)
        return carry

    cidx_prefetch(0, 0)
    round_(0, 0, drain=False)
    round_(1, 1, drain=True)
    lax.fori_loop(1, kmax // 2, body, 0)
    for b in range(NBUF):
        ch = chunk(kmax - 1, b)

        @pl.when(ch < lim)
        def _():
            scatter_wait(b)


def _zero_acc(acc, gbuf, s):
    zero = jnp.zeros((16,), jnp.float32)

    def zrow(r, carry):
        for l in range(D // 16):
            gbuf[0, r, pl.ds(l * 16, 16)] = zero
        return carry

    lax.fori_loop(0, RB, zrow, 0)
    for k in range((NRC + NS - 1) // NS):
        ch = s + k * NS

        @pl.when(ch < NRC)
        def _():
            pltpu.sync_copy(gbuf.at[0, pl.ds(0, RB)], acc.at[pl.ds(ch * RB, RB)])


def _writeback(acc, s, copy_out):
    for k in range((NRC + NS - 1) // NS):
        ch = s + k * NS

        @pl.when(ch < NRC)
        def _():
            r0 = ch * RB
            copy_out(acc.at[pl.ds(r0, RB)], r0)


def _body(x_hbm, row1, col1, row2, col2, out_hbm, x2b_hbm,
          acc, ridx, cidx, gbuf, irs, ics, gs, ss):
    c = lax.axis_index("c")
    s = lax.axis_index("s")

    _zero_acc(acc, gbuf, s)
    plsc.subcore_barrier()

    @pl.when(c == 0)
    def _():
        _spmm_edges(row1, col1, x_hbm, acc, ridx, cidx, gbuf,
                    irs, ics, gs, ss, s, 0, NCH1)
        plsc.subcore_barrier()
        _writeback(acc, s,
                   lambda src, r0: pltpu.sync_copy(
                       src, out_hbm.at[pl.ds(r0, RB), pl.ds(0, D)]))
        plsc.subcore_barrier()
        _zero_acc(acc, gbuf, s)
        plsc.subcore_barrier()
        _spmm_edges(row2, col2, x_hbm, acc, ridx, cidx, gbuf,
                    irs, ics, gs, ss, s, NCH2A, NCH2B)
        plsc.subcore_barrier()
        _writeback(acc, s,
                   lambda src, r0: pltpu.sync_copy(
                       src, x2b_hbm.at[pl.ds(r0, RB)]))

    @pl.when(c == 1)
    def _():
        _spmm_edges(row2, col2, x_hbm, acc, ridx, cidx, gbuf,
                    irs, ics, gs, ss, s, 0, NCH2A)
        plsc.subcore_barrier()
        _writeback(acc, s,
                   lambda src, r0: pltpu.sync_copy(
                       src, out_hbm.at[pl.ds(r0, RB), pl.ds(D, D)]))


def _merge_body(part_ref, x2b_ref, out_ref):
    out_ref[...] = part_ref[...] + x2b_ref[...]


@jax.jit
def kernel(x, adj_t, adj_t2):
    mesh = plsc.VectorSubcoreMesh(core_axis_name="c", subcore_axis_name="s")
    f = pl.kernel(
        _body,
        out_type=(jax.ShapeDtypeStruct((N, 2 * D), jnp.float32),
                  jax.ShapeDtypeStruct((N, D), jnp.float32)),
        mesh=mesh,
        scratch_types=[
            pltpu.VMEM_SHARED((N, D), jnp.float32),
            pltpu.VMEM((NBUF, CHUNK), jnp.int32),
            pltpu.VMEM((2 * NBUF, CHUNK), jnp.int32),
            pltpu.VMEM((NBUF, CHUNK, D), jnp.float32),
            pltpu.SemaphoreType.DMA((NBUF,)),
            pltpu.SemaphoreType.DMA((2 * NBUF,)),
            pltpu.SemaphoreType.DMA((NBUF,)),
            pltpu.SemaphoreType.DMA((NBUF,)),
        ],
    )
    part, x2b = f(x, adj_t[0], adj_t[1], adj_t2[0], adj_t2[1])
    nblk = 10
    return pl.pallas_call(
        _merge_body,
        out_shape=jax.ShapeDtypeStruct((N, 2 * D), jnp.float32),
        grid=(nblk,),
        in_specs=[
            pl.BlockSpec((N // nblk, D), lambda i: (i, 1)),
            pl.BlockSpec((N // nblk, D), lambda i: (i, 0)),
        ],
        out_specs=pl.BlockSpec((N // nblk, D), lambda i: (i, 1)),
        input_output_aliases={0: 0},
    )(part, x2b)


# speedup vs baseline: 1.0474x; 1.0474x over previous
import jax
import jax.numpy as jnp
from jax import lax
from jax.experimental import pallas as pl
from jax.experimental.pallas import tpu as pltpu
from jax.experimental.pallas import tpu_sc as plsc

N = 10000
D = 128
E1 = 320000
E2 = 640000
NS = 16
CHUNK = 128
NBUF = 3
RB = 80
NRC = N // RB
NCH1 = E1 // CHUNK
NCH2 = E2 // CHUNK
NCH2A = 3792
NCH2B = NCH2 - NCH2A


def _spmm_edges(row_hbm, col_hbm, x_hbm, acc, ridx, cidx, gbuf,
                irs, ics, gs, ss, s, ch0, nch):
    kmax = (nch // NS + NBUF) // NBUF
    kmax += kmax % 2
    lim = ch0 + nch

    def chunk(k, b):
        return ch0 + s + (k * NBUF + b) * NS

    def scatter_wait(b):
        pltpu.make_async_copy(gbuf.at[b], acc.at[ridx.at[b]], ss.at[b]).wait()

    def cidx_prefetch(k, q):
        for b in range(NBUF):
            ch = chunk(k, b)

            @pl.when(ch < lim)
            def _():
                i = q * NBUF + b
                pltpu.async_copy(col_hbm.at[pl.ds(ch * CHUNK, CHUNK)],
                                 cidx.at[i], ics.at[i])

    def round_(k, p, drain):
        if drain:
            for b in range(NBUF):
                chp = chunk(k - 1, b)

                @pl.when(chp < lim)
                def _():
                    scatter_wait(b)
        cidx_prefetch(k + 1, 1 - p)
        for b in range(NBUF):
            ch = chunk(k, b)

            @pl.when(ch < lim)
            def _():
                i = p * NBUF + b
                off = ch * CHUNK
                pltpu.async_copy(row_hbm.at[pl.ds(off, CHUNK)], ridx.at[b],
                                 irs.at[b])
                pltpu.make_async_copy(col_hbm.at[pl.ds(off, CHUNK)],
                                      cidx.at[i], ics.at[i]).wait()
                pltpu.async_copy(x_hbm.at[cidx.at[i]], gbuf.at[b], gs.at[b])
        for b in range(NBUF):
            ch = chunk(k, b)

            @pl.when(ch < lim)
            def _():
                i = p * NBUF + b
                off = ch * CHUNK
                pltpu.make_async_copy(row_hbm.at[pl.ds(off, CHUNK)],
                                      ridx.at[b], irs.at[b]).wait()
                pltpu.make_async_copy(x_hbm.at[cidx.at[i]], gbuf.at[b],
                                      gs.at[b]).wait()
                pltpu.async_copy(gbuf.at[b], acc.at[ridx.at[b]], ss.at[b],
                                 add=True)

    def body(m, carry):
        k = m * 2
        round_(k, 0, drain=True)
        round_(k + 1, 1, drain=True)
        return carry

    cidx_prefetch(0, 0)
    round_(0, 0, drain=False)
    round_(1, 1, drain=True)
    lax.fori_loop(1, kmax // 2, body, 0)
    for b in range(NBUF):
        ch = chunk(kmax - 1, b)

        @pl.when(ch < lim)
        def _():
            scatter_wait(b)


def _zero_acc(acc, gbuf, s):
    zero = jnp.zeros((16,), jnp.float32)

    def zrow(r, carry):
        for l in range(D // 16):
            gbuf[0, r, pl.ds(l * 16, 16)] = zero
        return carry

    lax.fori_loop(0, RB, zrow, 0)
    for k in range((NRC + NS - 1) // NS):
        ch = s + k * NS

        @pl.when(ch < NRC)
        def _():
            pltpu.sync_copy(gbuf.at[0, pl.ds(0, RB)], acc.at[pl.ds(ch * RB, RB)])


def _writeback(acc, s, copy_out):
    for k in range((NRC + NS - 1) // NS):
        ch = s + k * NS

        @pl.when(ch < NRC)
        def _():
            r0 = ch * RB
            copy_out(acc.at[pl.ds(r0, RB)], r0)


def _body(x_hbm, row1, col1, row2, col2, out_hbm, x2b_hbm,
          acc, ridx, cidx, gbuf, irs, ics, gs, ss):
    c = lax.axis_index("c")
    s = lax.axis_index("s")

    _zero_acc(acc, gbuf, s)
    plsc.subcore_barrier()

    @pl.when(c == 0)
    def _():
        _spmm_edges(row1, col1, x_hbm, acc, ridx, cidx, gbuf,
                    irs, ics, gs, ss, s, 0, NCH1)
        plsc.subcore_barrier()
        _writeback(acc, s,
                   lambda src, r0: pltpu.sync_copy(
                       src, out_hbm.at[pl.ds(r0, RB), pl.ds(0, D)]))
        plsc.subcore_barrier()
        _zero_acc(acc, gbuf, s)
        plsc.subcore_barrier()
        _spmm_edges(row2, col2, x_hbm, acc, ridx, cidx, gbuf,
                    irs, ics, gs, ss, s, NCH2A, NCH2B)
        plsc.subcore_barrier()
        _writeback(acc, s,
                   lambda src, r0: pltpu.sync_copy(
                       src, x2b_hbm.at[pl.ds(r0, RB)]))

    @pl.when(c == 1)
    def _():
        _spmm_edges(row2, col2, x_hbm, acc, ridx, cidx, gbuf,
                    irs, ics, gs, ss, s, 0, NCH2A)
        plsc.subcore_barrier()
        _writeback(acc, s,
                   lambda src, r0: pltpu.sync_copy(
                       src, out_hbm.at[pl.ds(r0, RB), pl.ds(D, D)]))


def _merge_body(part_ref, x2b_ref, out_ref):
    out_ref[...] = part_ref[...] + x2b_ref[...]


@jax.jit
def kernel(x, adj_t, adj_t2):
    mesh = plsc.VectorSubcoreMesh(core_axis_name="c", subcore_axis_name="s")
    f = pl.kernel(
        _body,
        out_type=(jax.ShapeDtypeStruct((N, 2 * D), jnp.float32),
                  jax.ShapeDtypeStruct((N, D), jnp.float32)),
        mesh=mesh,
        scratch_types=[
            pltpu.VMEM_SHARED((N, D), jnp.float32),
            pltpu.VMEM((NBUF, CHUNK), jnp.int32),
            pltpu.VMEM((2 * NBUF, CHUNK), jnp.int32),
            pltpu.VMEM((NBUF, CHUNK, D), jnp.float32),
            pltpu.SemaphoreType.DMA((NBUF,)),
            pltpu.SemaphoreType.DMA((2 * NBUF,)),
            pltpu.SemaphoreType.DMA((NBUF,)),
            pltpu.SemaphoreType.DMA((NBUF,)),
        ],
    )
    part, x2b = f(x, adj_t[0], adj_t[1], adj_t2[0], adj_t2[1])
    nblk = 10
    return pl.pallas_call(
        _merge_body,
        out_shape=jax.ShapeDtypeStruct((N, 2 * D), jnp.float32),
        grid=(nblk,),
        in_specs=[
            pl.BlockSpec((N // nblk, D), lambda i: (i, 1)),
            pl.BlockSpec((N // nblk, D), lambda i: (i, 0)),
        ],
        out_specs=pl.BlockSpec((N // nblk, D), lambda i: (i, 1)),
        input_output_aliases={0: 0},
    )(part, x2b)
